# baseline (speedup 1.0000x reference)
import functools

import jax
import jax.numpy as jnp
from jax.experimental import pallas as pl
from jax.experimental.pallas import tpu as pltpu

_NX = 2
_NW = 4


def _gcn_body(x_hbm, w_hbm, b_ref, o_ref, *scratch, d_feat):
    xvs = scratch[:_NX]
    wvs = scratch[_NX:_NX + _NW]
    sems = scratch[_NX + _NW:]
    n = x_hbm.shape[0]
    out_dim = w_hbm.shape[0]
    xs = n // _NX
    ws = out_dim // _NW

    copies = []
    for k in range(_NX):
        c = pltpu.make_async_copy(x_hbm.at[pl.ds(k * xs, xs), :], xvs[k],
                                  sems[k])
        c.start()
        copies.append(c)
    w_copies = []
    for k in range(_NW):
        c = pltpu.make_async_copy(w_hbm.at[pl.ds(k * ws, ws), :], wvs[k],
                                  sems[_NX + k])
        c.start()
        w_copies.append(c)

    for c in copies:
        c.wait()
    hs = [jnp.sum(xv[...], axis=1, keepdims=True) * (1.0 / d_feat)
          for xv in xvs]
    h = jnp.concatenate(hs, axis=0)

    for k in range(_NW):
        w_copies[k].wait()
        part = jnp.dot(wvs[k][...], h, preferred_element_type=jnp.float32)
        o_ref[k * ws:(k + 1) * ws, :] = part + b_ref[k * ws:(k + 1) * ws, :]


def kernel(x, edge_index, W, b):
    del edge_index
    n, d = x.shape
    out_dim = W.shape[0]
    xs = n // _NX
    ws = out_dim // _NW

    body = functools.partial(_gcn_body, d_feat=d)
    out = pl.pallas_call(
        body,
        in_specs=[
            pl.BlockSpec(memory_space=pl.ANY),
            pl.BlockSpec(memory_space=pl.ANY),
            pl.BlockSpec((out_dim, 1), lambda: (0, 0)),
        ],
        out_specs=pl.BlockSpec((out_dim, 1), lambda: (0, 0)),
        out_shape=jax.ShapeDtypeStruct((out_dim, 1), jnp.float32),
        scratch_shapes=[pltpu.VMEM((xs, d), jnp.float32) for _ in range(_NX)]
        + [pltpu.VMEM((ws, n), jnp.float32) for _ in range(_NW)]
        + [pltpu.SemaphoreType.DMA for _ in range(_NX + _NW)],
    )(x, W, b.reshape(out_dim, 1))
    return out.reshape(out_dim)


# speedup vs baseline: 1.0744x; 1.0744x over previous
import functools

import jax
import jax.numpy as jnp
from jax.experimental import pallas as pl

def _gcn_body(x_ref, w_ref, b_ref, o_ref, *, d_feat):
    h = jnp.sum(x_ref[...], axis=1, keepdims=True) * (1.0 / d_feat)
    o_ref[...] = b_ref[...] + jnp.dot(w_ref[...], h,
                                      preferred_element_type=jnp.float32)


def kernel(x, edge_index, W, b):
    del edge_index
    n, d = x.shape
    out_dim = W.shape[0]

    body = functools.partial(_gcn_body, d_feat=d)
    out = pl.pallas_call(
        body,
        in_specs=[
            pl.BlockSpec((n, d), lambda: (0, 0)),
            pl.BlockSpec((out_dim, n), lambda: (0, 0)),
            pl.BlockSpec((out_dim, 1), lambda: (0, 0)),
        ],
        out_specs=pl.BlockSpec((out_dim, 1), lambda: (0, 0)),
        out_shape=jax.ShapeDtypeStruct((out_dim, 1), jnp.float32),
    )(x, W, b.reshape(out_dim, 1))
    return out.reshape(out_dim)
